# Initial kernel scaffold; baseline (speedup 1.0000x reference)
import jax
import jax.numpy as jnp
from jax.experimental import pallas as pl


def kernel(x, mat):
    raise NotImplementedError("write your pallas kernel here")


# speedup vs baseline: 261.6581x; 261.6581x over previous
import jax
import jax.numpy as jnp
from jax.experimental import pallas as pl

IN_DIM = 256
OUT_DIM = 128
B_BLK = 2048


def _tc_body(x_ref, mat_ref, o_ref):
    m = mat_ref[...]
    a1 = (m == 1).astype(jnp.float32)
    a2 = (m == 2).astype(jnp.float32)
    z = x_ref[...] * 0.5
    t = jnp.tanh(z)
    o_ref[...] = (
        jax.lax.dot_general(z, a1, (((1,), (1,)), ((), ())),
                            preferred_element_type=jnp.float32)
        + jax.lax.dot_general(t, a2, (((1,), (1,)), ((), ())),
                              preferred_element_type=jnp.float32)
    )


def kernel(x, mat):
    batch = x.shape[0]
    mat32 = mat.astype(jnp.int32)
    return pl.pallas_call(
        _tc_body,
        grid=(batch // B_BLK,),
        in_specs=[
            pl.BlockSpec((B_BLK, IN_DIM), lambda i: (i, 0)),
            pl.BlockSpec((OUT_DIM, IN_DIM), lambda i: (0, 0)),
        ],
        out_specs=pl.BlockSpec((B_BLK, OUT_DIM), lambda i: (i, 0)),
        out_shape=jax.ShapeDtypeStruct((batch, OUT_DIM), jnp.float32),
    )(x, mat32)
